# Initial kernel scaffold; baseline (speedup 1.0000x reference)
import jax
import jax.numpy as jnp
from jax.experimental import pallas as pl


def kernel(x, edge_index, query_kernel, query_bias, key_kernel, key_bias, kernel, bias):
    raise NotImplementedError("write your pallas kernel here")


# speedup vs baseline: 2.8115x; 2.8115x over previous
import functools

import jax
import jax.numpy as jnp
from jax import lax
from jax.experimental import pallas as pl
from jax.experimental.pallas import tpu as pltpu
from jax.experimental.pallas import tpu_sc as plsc

NC = 2
NS = 16
L = 16
NW = NC * NS
C = 128
WCOL = 144


def _qkv_tc(x_pad, wq, bq, wk, bk, wv):
    n_pad, d = x_pad.shape
    blk = 1024
    grid = n_pad // blk

    def body(x_ref, wq_ref, bq_ref, wk_ref, bk_ref, wv_ref, q_ref, k_ref, v_ref):
        xb = x_ref[...]
        q_ref[...] = jnp.maximum(
            jnp.dot(xb, wq_ref[...], preferred_element_type=jnp.float32) + bq_ref[...], 0.0)
        k_ref[...] = jnp.maximum(
            jnp.dot(xb, wk_ref[...], preferred_element_type=jnp.float32) + bk_ref[...], 0.0)
        v_ref[...] = jnp.dot(xb, wv_ref[...], preferred_element_type=jnp.float32)

    return pl.pallas_call(
        body,
        grid=(grid,),
        in_specs=[
            pl.BlockSpec((blk, d), lambda i: (i, 0)),
            pl.BlockSpec((d, d), lambda i: (0, 0)),
            pl.BlockSpec((1, d), lambda i: (0, 0)),
            pl.BlockSpec((d, d), lambda i: (0, 0)),
            pl.BlockSpec((1, d), lambda i: (0, 0)),
            pl.BlockSpec((d, d), lambda i: (0, 0)),
        ],
        out_specs=[pl.BlockSpec((blk, d), lambda i: (i, 0))] * 3,
        out_shape=[jax.ShapeDtypeStruct((n_pad, d), jnp.float32)] * 3,
    )(x_pad, wq, bq, wk, bk, wv)


def _max_tc(s):
    rows = s.shape[0] // 128
    s2 = s.reshape(rows, 128)

    def body(s_ref, m_ref):
        m_ref[...] = jnp.full((8, 128), jnp.max(s_ref[...]), jnp.float32)

    return pl.pallas_call(
        body, out_shape=jax.ShapeDtypeStruct((8, 128), jnp.float32))(s2)


def _scores_sc(q, k, row, col):
    n_pad, d = q.shape
    e_pad = row.shape[0]
    per_w = e_pad // NW
    n_chunks = per_w // C
    mesh = plsc.VectorSubcoreMesh(
        core_axis_name="c", subcore_axis_name="s", num_cores=NC, num_subcores=NS)

    @functools.partial(
        pl.kernel,
        out_type=jax.ShapeDtypeStruct((e_pad,), jnp.float32),
        mesh=mesh,
        compiler_params=pltpu.CompilerParams(
            needs_layout_passes=False, use_tc_tiling_on_sc=False),
        scratch_types=[
            pltpu.VMEM((C,), jnp.int32),
            pltpu.VMEM((C,), jnp.int32),
            pltpu.VMEM((C, d), jnp.float32),
            pltpu.VMEM((C, d), jnp.float32),
            pltpu.VMEM((C,), jnp.float32),
            pltpu.SemaphoreType.DMA,
            pltpu.SemaphoreType.DMA,
        ],
    )
    def scores(q_hbm, k_hbm, row_hbm, col_hbm, s_hbm,
               row_v, col_v, qrows, krows, s_v, sem_q, sem_k):
        wid = lax.axis_index("s") * NC + lax.axis_index("c")
        base0 = wid * per_w

        def chunk(i, carry):
            base = base0 + i * C
            pltpu.sync_copy(row_hbm.at[pl.ds(base, C)], row_v)
            pltpu.sync_copy(col_hbm.at[pl.ds(base, C)], col_v)
            cq = pltpu.async_copy(q_hbm.at[row_v], qrows, sem_q)
            ck = pltpu.async_copy(k_hbm.at[col_v], krows, sem_k)
            cq.wait()
            ck.wait()

            def group(g, c2):
                e16 = g * L + lax.iota(jnp.int32, L)
                dvec = jnp.zeros((L,), jnp.int32)
                accs = [jnp.zeros((L,), jnp.float32) for _ in range(4)]
                for dd in range(d):
                    qv = plsc.load_gather(qrows, [e16, dvec])
                    kv = plsc.load_gather(krows, [e16, dvec])
                    accs[dd % 4] = accs[dd % 4] + qv * kv
                    dvec = dvec + 1
                s_v[pl.ds(g * L, L)] = (accs[0] + accs[1]) + (accs[2] + accs[3])
                return c2

            lax.fori_loop(0, C // L, group, 0)
            pltpu.sync_copy(s_v, s_hbm.at[pl.ds(base, C)])
            return carry

        lax.fori_loop(0, n_chunks, chunk, 0)

    return scores(q, k, row, col)


def _accum_sc(v, s, m, row, col):
    n_pad, d = v.shape
    e_pad = row.shape[0]
    per_w = e_pad // NW
    n_chunks = per_w // C
    rows_per_tile = n_pad // NS
    mesh = plsc.VectorSubcoreMesh(
        core_axis_name="c", subcore_axis_name="s", num_cores=NC, num_subcores=NS)

    @functools.partial(
        pl.kernel,
        out_type=jax.ShapeDtypeStruct((NC * n_pad, WCOL), jnp.float32),
        mesh=mesh,
        compiler_params=pltpu.CompilerParams(
            needs_layout_passes=False, use_tc_tiling_on_sc=False),
        scratch_types=[
            pltpu.VMEM((C,), jnp.int32),
            pltpu.VMEM((C,), jnp.int32),
            pltpu.VMEM((C, d), jnp.float32),
            pltpu.VMEM((C, WCOL), jnp.float32),
            pltpu.VMEM((C,), jnp.float32),
            pltpu.VMEM((L,), jnp.float32),
            pltpu.VMEM_SHARED((n_pad, WCOL), jnp.float32),
            pltpu.SemaphoreType.DMA,
        ],
    )
    def accum(v_hbm, s_hbm, m_hbm, row_hbm, col_hbm, out_hbm,
              row_v, col_v, vrows, sbuf, s_v, m_v, hacc, sem_v):
        cid = lax.axis_index("c")
        sid = lax.axis_index("s")
        wid = sid * NC + cid

        def zrow(r, c0):
            for c9 in range(WCOL // L):
                sbuf[r, pl.ds(c9 * L, L)] = jnp.zeros((L,), jnp.float32)
            return c0

        lax.fori_loop(0, C, zrow, 0)
        for b in range(rows_per_tile // 128):
            pltpu.sync_copy(sbuf, hacc.at[pl.ds(sid * rows_per_tile + b * 128, 128)])
        plsc.subcore_barrier()

        pltpu.sync_copy(m_hbm.at[pl.ds(0, L)], m_v)
        mvec = m_v[...]
        base0 = wid * per_w

        def chunk(i, carry):
            base = base0 + i * C
            pltpu.sync_copy(row_hbm.at[pl.ds(base, C)], row_v)
            pltpu.sync_copy(col_hbm.at[pl.ds(base, C)], col_v)
            pltpu.sync_copy(s_hbm.at[pl.ds(base, C)], s_v)
            cv = pltpu.async_copy(v_hbm.at[col_v], vrows, sem_v)
            cv.wait()

            def group(g, c2):
                e16 = g * L + lax.iota(jnp.int32, L)
                s16 = s_v[pl.ds(g * L, L)]
                w16 = jnp.exp(s16 - mvec)
                dvec = jnp.zeros((L,), jnp.int32)
                for dd in range(d):
                    val = plsc.load_gather(vrows, [e16, dvec]) * w16
                    plsc.store_scatter(sbuf, [e16, dvec], val)
                    dvec = dvec + 1
                for dd in range(WCOL - d):
                    plsc.store_scatter(sbuf, [e16, dvec], w16)
                    dvec = dvec + 1
                return c2

            lax.fori_loop(0, C // L, group, 0)
            pltpu.sync_copy(sbuf, hacc.at[row_v], add=True)
            return carry

        lax.fori_loop(0, n_chunks, chunk, 0)
        plsc.subcore_barrier()
        for b in range(rows_per_tile // 128):
            off = sid * rows_per_tile + b * 128
            pltpu.sync_copy(hacc.at[pl.ds(off, 128)],
                            out_hbm.at[pl.ds(cid * n_pad + off, 128)])

    return accum(v, s, m, row, col)


def _finalize_tc(acc, bias2d, n_pad):
    blk = 1024
    grid = n_pad // blk

    def body(a0_ref, a1_ref, b_ref, h_ref):
        tot = a0_ref[...] + a1_ref[...]
        h_ref[...] = tot[:, :128] / tot[:, 128:129] + b_ref[...]

    return pl.pallas_call(
        body,
        grid=(grid,),
        in_specs=[
            pl.BlockSpec((blk, WCOL), lambda i: (i, 0)),
            pl.BlockSpec((blk, WCOL), lambda i, g=grid: (i + g, 0)),
            pl.BlockSpec((1, 128), lambda i: (0, 0)),
        ],
        out_specs=pl.BlockSpec((blk, 128), lambda i: (i, 0)),
        out_shape=jax.ShapeDtypeStruct((n_pad, 128), jnp.float32),
    )(acc, acc, bias2d)


def kernel(x, edge_index, query_kernel, query_bias, key_kernel, key_bias, kernel, bias):
    n, d = x.shape
    e = edge_index.shape[1]
    n_pad = -(-n // 2048) * 2048
    e_tot = e + n
    e_pad = -(-e_tot // (NW * C)) * (NW * C)

    loop = jnp.arange(n, dtype=jnp.int32)
    padidx = jnp.full((e_pad - e_tot,), n, dtype=jnp.int32)
    row = jnp.concatenate([edge_index[0], loop, padidx])
    col = jnp.concatenate([edge_index[1], loop, padidx])
    x_pad = jnp.pad(x, ((0, n_pad - n), (0, 0)))

    q, k, v = _qkv_tc(x_pad, query_kernel, query_bias.reshape(1, -1),
                      key_kernel, key_bias.reshape(1, -1), kernel)
    s = _scores_sc(q, k, row, col)
    m = _max_tc(s).reshape(-1)
    acc = _accum_sc(v, s, m, row, col)
    h = _finalize_tc(acc, bias.reshape(1, -1), n_pad)
    return h[:n]


# speedup vs baseline: 3.2502x; 1.1560x over previous
import functools

import jax
import jax.numpy as jnp
from jax import lax
from jax.experimental import pallas as pl
from jax.experimental.pallas import tpu as pltpu
from jax.experimental.pallas import tpu_sc as plsc

NC = 2
NS = 16
L = 16
NW = NC * NS
C = 128
HW = 64
WCOL = HW + L

_SC_PARAMS = pltpu.CompilerParams(
    needs_layout_passes=False, use_tc_tiling_on_sc=False)


def _qkv_tc(x_pad, wq, bq, wk, bk, wv):
    n_pad, d = x_pad.shape
    blk = 1024
    grid = n_pad // blk

    def body(x_ref, wq_ref, bq_ref, wk_ref, bk_ref, wv_ref, q_ref, k_ref, v_ref):
        xb = x_ref[...]
        q_ref[...] = jnp.maximum(
            jnp.dot(xb, wq_ref[...], preferred_element_type=jnp.float32) + bq_ref[...], 0.0)
        k_ref[...] = jnp.maximum(
            jnp.dot(xb, wk_ref[...], preferred_element_type=jnp.float32) + bk_ref[...], 0.0)
        v_ref[...] = jnp.dot(xb, wv_ref[...], preferred_element_type=jnp.float32)

    return pl.pallas_call(
        body,
        grid=(grid,),
        in_specs=[
            pl.BlockSpec((blk, d), lambda i: (i, 0)),
            pl.BlockSpec((d, d), lambda i: (0, 0)),
            pl.BlockSpec((1, d), lambda i: (0, 0)),
            pl.BlockSpec((d, d), lambda i: (0, 0)),
            pl.BlockSpec((1, d), lambda i: (0, 0)),
            pl.BlockSpec((d, d), lambda i: (0, 0)),
        ],
        out_specs=[pl.BlockSpec((blk, d), lambda i: (i, 0))] * 3,
        out_shape=[jax.ShapeDtypeStruct((n_pad, d), jnp.float32)] * 3,
    )(x_pad, wq, bq, wk, bk, wv)


def _max_tc(s):
    rows = s.shape[0] // 128
    s2 = s.reshape(rows, 128)

    def body(s_ref, m_ref):
        m_ref[...] = jnp.full((8, 128), jnp.max(s_ref[...]), jnp.float32)

    return pl.pallas_call(
        body, out_shape=jax.ShapeDtypeStruct((8, 128), jnp.float32))(s2)


def _scores_sc(q, k, row, col):
    n_pad, d = q.shape
    e_pad = row.shape[0]
    per_w = e_pad // NW
    n_chunks = per_w // C
    n_pairs = n_chunks // 2
    mesh = plsc.VectorSubcoreMesh(
        core_axis_name="c", subcore_axis_name="s", num_cores=NC, num_subcores=NS)

    @functools.partial(
        pl.kernel,
        out_type=jax.ShapeDtypeStruct((e_pad,), jnp.float32),
        mesh=mesh,
        compiler_params=_SC_PARAMS,
        scratch_types=[
            [pltpu.VMEM((C,), jnp.int32)] * 2,
            [pltpu.VMEM((C,), jnp.int32)] * 2,
            [pltpu.VMEM((C, d), jnp.float32)] * 2,
            [pltpu.VMEM((C, d), jnp.float32)] * 2,
            [pltpu.VMEM((C,), jnp.float32)] * 2,
            [pltpu.SemaphoreType.DMA] * 2,
            [pltpu.SemaphoreType.DMA] * 2,
            [pltpu.SemaphoreType.DMA] * 2,
            [pltpu.SemaphoreType.DMA] * 2,
            [pltpu.SemaphoreType.DMA] * 2,
        ],
    )
    def scores(q_hbm, k_hbm, row_hbm, col_hbm, s_hbm,
               row_v, col_v, qr, kr, sv, sem_ir, sem_ic, sem_q, sem_k, sem_s):
        wid = lax.axis_index("s") * NC + lax.axis_index("c")
        base0 = wid * per_w

        def issue_idx(c, p):
            base = base0 + c * C
            pltpu.async_copy(row_hbm.at[pl.ds(base, C)], row_v[p], sem_ir[p])
            pltpu.async_copy(col_hbm.at[pl.ds(base, C)], col_v[p], sem_ic[p])

        def wait_idx(p):
            pltpu.make_async_copy(row_hbm.at[pl.ds(0, C)], row_v[p], sem_ir[p]).wait()
            pltpu.make_async_copy(col_hbm.at[pl.ds(0, C)], col_v[p], sem_ic[p]).wait()

        def issue_gather(p):
            pltpu.async_copy(q_hbm.at[row_v[p]], qr[p], sem_q[p])
            pltpu.async_copy(k_hbm.at[col_v[p]], kr[p], sem_k[p])

        def wait_gather(p):
            pltpu.make_async_copy(q_hbm.at[row_v[p]], qr[p], sem_q[p]).wait()
            pltpu.make_async_copy(k_hbm.at[col_v[p]], kr[p], sem_k[p]).wait()

        def compute(p):
            def group(g, c2):
                e16 = g * L + lax.iota(jnp.int32, L)
                dvec = jnp.zeros((L,), jnp.int32)
                accs = [jnp.zeros((L,), jnp.float32) for _ in range(4)]
                for dd in range(d):
                    qv = plsc.load_gather(qr[p], [e16, dvec])
                    kv = plsc.load_gather(kr[p], [e16, dvec])
                    accs[dd % 4] = accs[dd % 4] + qv * kv
                    dvec = dvec + 1
                sv[p][pl.ds(g * L, L)] = (accs[0] + accs[1]) + (accs[2] + accs[3])
                return c2

            lax.fori_loop(0, C // L, group, 0)

        def chunk_body(c, p):
            wait_gather(p)

            @pl.when(c >= 2)
            def _():
                pltpu.make_async_copy(sv[p], s_hbm.at[pl.ds(base0, C)], sem_s[p]).wait()

            @pl.when(c + 2 < n_chunks)
            def _():
                issue_idx(c + 2, p)

            @pl.when(c + 1 < n_chunks)
            def _():
                wait_idx(1 - p)
                issue_gather(1 - p)

            compute(p)
            pltpu.async_copy(sv[p], s_hbm.at[pl.ds(base0 + c * C, C)], sem_s[p])

        issue_idx(0, 0)
        issue_idx(1, 1)
        wait_idx(0)
        issue_gather(0)

        def pair(j, carry):
            chunk_body(2 * j, 0)
            chunk_body(2 * j + 1, 1)
            return carry

        lax.fori_loop(0, n_pairs, pair, 0)
        pltpu.make_async_copy(sv[0], s_hbm.at[pl.ds(base0, C)], sem_s[0]).wait()
        pltpu.make_async_copy(sv[1], s_hbm.at[pl.ds(base0, C)], sem_s[1]).wait()

    return scores(q, k, row, col)


def _accum_sc(v, s, m, row, col):
    n_pad, d = v.shape
    e_pad = row.shape[0]
    per_s = e_pad // NS
    n_chunks = per_s // C
    n_pairs = n_chunks // 2
    rows_per_tile = n_pad // NS
    mesh = plsc.VectorSubcoreMesh(
        core_axis_name="c", subcore_axis_name="s", num_cores=NC, num_subcores=NS)

    @functools.partial(
        pl.kernel,
        out_type=jax.ShapeDtypeStruct((NC * n_pad, WCOL), jnp.float32),
        mesh=mesh,
        compiler_params=_SC_PARAMS,
        scratch_types=[
            [pltpu.VMEM((C,), jnp.int32)] * 2,
            [pltpu.VMEM((C,), jnp.int32)] * 2,
            [pltpu.VMEM((C,), jnp.int32)] * 2,
            [pltpu.VMEM((C,), jnp.float32)] * 2,
            pltpu.VMEM((C,), jnp.float32),
            [pltpu.VMEM((C, d), jnp.float32)] * 2,
            [pltpu.VMEM((C, WCOL), jnp.float32)] * 2,
            pltpu.VMEM((L,), jnp.float32),
            pltpu.VMEM_SHARED((n_pad, WCOL), jnp.float32),
            [pltpu.SemaphoreType.DMA] * 2,
            [pltpu.SemaphoreType.DMA] * 2,
            [pltpu.SemaphoreType.DMA] * 2,
            [pltpu.SemaphoreType.DMA] * 2,
            [pltpu.SemaphoreType.DMA] * 2,
        ],
    )
    def accum(v_hbm, s_hbm, m_hbm, row_hbm, col_hbm, out_hbm,
              row_v, col_v, rowsc, s_v, scmp, vrows, sbuf, m_v, hacc,
              sem_ir, sem_ic, sem_is, sem_v, sem_sc):
        cid = lax.axis_index("c")
        sid = lax.axis_index("s")

        def zrow(r, c0):
            for c9 in range(WCOL // L):
                sbuf[0][r, pl.ds(c9 * L, L)] = jnp.zeros((L,), jnp.float32)
            return c0

        lax.fori_loop(0, C, zrow, 0)
        for b in range(rows_per_tile // 128):
            pltpu.sync_copy(sbuf[0], hacc.at[pl.ds(sid * rows_per_tile + b * 128, 128)])
        plsc.subcore_barrier()

        pltpu.sync_copy(m_hbm.at[pl.ds(0, L)], m_v)
        mvec = m_v[...]
        dbase = cid * HW
        base0 = sid * per_s

        def issue_idx(c, p):
            base = base0 + c * C
            pltpu.async_copy(row_hbm.at[pl.ds(base, C)], row_v[p], sem_ir[p])
            pltpu.async_copy(col_hbm.at[pl.ds(base, C)], col_v[p], sem_ic[p])
            pltpu.async_copy(s_hbm.at[pl.ds(base, C)], s_v[p], sem_is[p])

        def chunk_body(c, p):
            pltpu.make_async_copy(v_hbm.at[col_v[p]], vrows[p], sem_v[p]).wait()
            pltpu.make_async_copy(row_hbm.at[pl.ds(0, C)], row_v[p], sem_ir[p]).wait()
            pltpu.make_async_copy(s_hbm.at[pl.ds(0, C)], s_v[p], sem_is[p]).wait()

            @pl.when(c >= 2)
            def _():
                pltpu.make_async_copy(sbuf[p], hacc.at[rowsc[p]], sem_sc[p]).wait()

            def stage(g, c0):
                rowsc[p][pl.ds(g * L, L)] = row_v[p][pl.ds(g * L, L)]
                scmp[pl.ds(g * L, L)] = s_v[p][pl.ds(g * L, L)]
                return c0

            lax.fori_loop(0, C // L, stage, 0)

            @pl.when(c + 2 < n_chunks)
            def _():
                issue_idx(c + 2, p)

            @pl.when(c + 1 < n_chunks)
            def _():
                pltpu.make_async_copy(col_hbm.at[pl.ds(0, C)], col_v[1 - p], sem_ic[1 - p]).wait()
                pltpu.async_copy(v_hbm.at[col_v[1 - p]], vrows[1 - p], sem_v[1 - p])

            def group(g, c0):
                e16 = g * L + lax.iota(jnp.int32, L)
                s16 = scmp[pl.ds(g * L, L)]
                w16 = jnp.exp(s16 - mvec)
                svec = jnp.full((L,), dbase, jnp.int32)
                dvec = jnp.zeros((L,), jnp.int32)
                for dd in range(HW):
                    val = plsc.load_gather(vrows[p], [e16, svec]) * w16
                    plsc.store_scatter(sbuf[p], [e16, dvec], val)
                    svec = svec + 1
                    dvec = dvec + 1
                for dd in range(L):
                    plsc.store_scatter(sbuf[p], [e16, dvec], w16)
                    dvec = dvec + 1
                return c0

            lax.fori_loop(0, C // L, group, 0)
            pltpu.async_copy(sbuf[p], hacc.at[rowsc[p]], sem_sc[p], add=True)

        issue_idx(0, 0)
        issue_idx(1, 1)
        pltpu.make_async_copy(col_hbm.at[pl.ds(0, C)], col_v[0], sem_ic[0]).wait()
        pltpu.async_copy(v_hbm.at[col_v[0]], vrows[0], sem_v[0])

        def pair(j, carry):
            chunk_body(2 * j, 0)
            chunk_body(2 * j + 1, 1)
            return carry

        lax.fori_loop(0, n_pairs, pair, 0)
        pltpu.make_async_copy(sbuf[0], hacc.at[rowsc[0]], sem_sc[0]).wait()
        pltpu.make_async_copy(sbuf[1], hacc.at[rowsc[1]], sem_sc[1]).wait()
        plsc.subcore_barrier()
        for b in range(rows_per_tile // 128):
            off = sid * rows_per_tile + b * 128
            pltpu.sync_copy(hacc.at[pl.ds(off, 128)],
                            out_hbm.at[pl.ds(cid * n_pad + off, 128)])

    return accum(v, s, m, row, col)


def _finalize_tc(acc, bias2d, n_pad):
    blk = 1024
    grid = n_pad // blk

    def body(a0_ref, a1_ref, b_ref, h_ref):
        a0 = a0_ref[...]
        a1 = a1_ref[...]
        h0 = a0[:, :HW] / a0[:, HW:HW + 1]
        h1 = a1[:, :HW] / a1[:, HW:HW + 1]
        h_ref[...] = jnp.concatenate([h0, h1], axis=1) + b_ref[...]

    return pl.pallas_call(
        body,
        grid=(grid,),
        in_specs=[
            pl.BlockSpec((blk, WCOL), lambda i: (i, 0)),
            pl.BlockSpec((blk, WCOL), lambda i, g=grid: (i + g, 0)),
            pl.BlockSpec((1, 128), lambda i: (0, 0)),
        ],
        out_specs=pl.BlockSpec((blk, 128), lambda i: (i, 0)),
        out_shape=jax.ShapeDtypeStruct((n_pad, 128), jnp.float32),
    )(acc, acc, bias2d)


def kernel(x, edge_index, query_kernel, query_bias, key_kernel, key_bias, kernel, bias):
    n, d = x.shape
    e = edge_index.shape[1]
    n_pad = -(-n // 2048) * 2048
    e_tot = e + n
    e_pad = -(-e_tot // (NW * C * 2)) * (NW * C * 2)

    loop = jnp.arange(n, dtype=jnp.int32)
    padidx = jnp.full((e_pad - e_tot,), n, dtype=jnp.int32)
    row = jnp.concatenate([edge_index[0], loop, padidx])
    col = jnp.concatenate([edge_index[1], loop, padidx])
    x_pad = jnp.pad(x, ((0, n_pad - n), (0, 0)))

    q, k, v = _qkv_tc(x_pad, query_kernel, query_bias.reshape(1, -1),
                      key_kernel, key_bias.reshape(1, -1), kernel)
    s = _scores_sc(q, k, row, col)
    m = _max_tc(s).reshape(-1)
    acc = _accum_sc(v, s, m, row, col)
    h = _finalize_tc(acc, bias.reshape(1, -1), n_pad)
    return h[:n]


# speedup vs baseline: 6.9249x; 2.1306x over previous
import functools

import jax
import jax.numpy as jnp
from jax import lax
from jax.experimental import pallas as pl
from jax.experimental.pallas import tpu as pltpu
from jax.experimental.pallas import tpu_sc as plsc

NC = 2
NS = 16
L = 16
NW = NC * NS
C = 128
HW = 64
WCOL = HW + L

_SC_PARAMS = pltpu.CompilerParams(
    needs_layout_passes=False, use_tc_tiling_on_sc=False)


def _qkv_tc(x_pad, wq, bq, wk, bk, wv):
    n_pad, d = x_pad.shape
    blk = 1024
    grid = n_pad // blk

    def body(x_ref, wq_ref, bq_ref, wk_ref, bk_ref, wv_ref, q_ref, k_ref, v_ref):
        xb = x_ref[...]
        q_ref[...] = jnp.maximum(
            jnp.dot(xb, wq_ref[...], preferred_element_type=jnp.float32) + bq_ref[...], 0.0)
        k_ref[...] = jnp.maximum(
            jnp.dot(xb, wk_ref[...], preferred_element_type=jnp.float32) + bk_ref[...], 0.0)
        v_ref[...] = jnp.dot(xb, wv_ref[...], preferred_element_type=jnp.float32)

    return pl.pallas_call(
        body,
        grid=(grid,),
        in_specs=[
            pl.BlockSpec((blk, d), lambda i: (i, 0)),
            pl.BlockSpec((d, d), lambda i: (0, 0)),
            pl.BlockSpec((1, d), lambda i: (0, 0)),
            pl.BlockSpec((d, d), lambda i: (0, 0)),
            pl.BlockSpec((1, d), lambda i: (0, 0)),
            pl.BlockSpec((d, d), lambda i: (0, 0)),
        ],
        out_specs=[pl.BlockSpec((blk, d), lambda i: (i, 0))] * 3,
        out_shape=[jax.ShapeDtypeStruct((n_pad, d), jnp.float32)] * 3,
    )(x_pad, wq, bq, wk, bk, wv)


def _max_tc(s):
    rows = s.shape[0] // 128
    s2 = s.reshape(rows, 128)

    def body(s_ref, m_ref):
        m_ref[...] = jnp.full((8, 128), jnp.max(s_ref[...]), jnp.float32)

    return pl.pallas_call(
        body, out_shape=jax.ShapeDtypeStruct((8, 128), jnp.float32))(s2)


def _scores_sc(q, k, row, col):
    n_pad, d = q.shape
    e_pad = row.shape[0]
    per_w = e_pad // NW
    n_chunks = per_w // C
    n_pairs = n_chunks // 2
    mesh = plsc.VectorSubcoreMesh(
        core_axis_name="c", subcore_axis_name="s", num_cores=NC, num_subcores=NS)

    @functools.partial(
        pl.kernel,
        out_type=jax.ShapeDtypeStruct((e_pad,), jnp.float32),
        mesh=mesh,
        compiler_params=_SC_PARAMS,
        scratch_types=[
            [pltpu.VMEM((C,), jnp.int32)] * 2,
            [pltpu.VMEM((C,), jnp.int32)] * 2,
            [pltpu.VMEM((C, d), jnp.float32)] * 2,
            [pltpu.VMEM((C, d), jnp.float32)] * 2,
            [pltpu.VMEM((C,), jnp.float32)] * 2,
            [pltpu.SemaphoreType.DMA] * 2,
            [pltpu.SemaphoreType.DMA] * 2,
            [pltpu.SemaphoreType.DMA] * 2,
            [pltpu.SemaphoreType.DMA] * 2,
            [pltpu.SemaphoreType.DMA] * 2,
        ],
    )
    def scores(q_hbm, k_hbm, row_hbm, col_hbm, s_hbm,
               row_v, col_v, qr, kr, sv, sem_ir, sem_ic, sem_q, sem_k, sem_s):
        wid = lax.axis_index("s") * NC + lax.axis_index("c")
        base0 = wid * per_w

        def issue_idx(c, p):
            base = base0 + c * C
            pltpu.async_copy(row_hbm.at[pl.ds(base, C)], row_v[p], sem_ir[p])
            pltpu.async_copy(col_hbm.at[pl.ds(base, C)], col_v[p], sem_ic[p])

        def wait_idx(p):
            pltpu.make_async_copy(row_hbm.at[pl.ds(0, C)], row_v[p], sem_ir[p]).wait()
            pltpu.make_async_copy(col_hbm.at[pl.ds(0, C)], col_v[p], sem_ic[p]).wait()

        def issue_gather(p):
            pltpu.async_copy(q_hbm.at[row_v[p]], qr[p], sem_q[p])
            pltpu.async_copy(k_hbm.at[col_v[p]], kr[p], sem_k[p])

        def wait_gather(p):
            pltpu.make_async_copy(q_hbm.at[row_v[p]], qr[p], sem_q[p]).wait()
            pltpu.make_async_copy(k_hbm.at[col_v[p]], kr[p], sem_k[p]).wait()

        def compute(p):
            def group(g, c2):
                lanes = lax.iota(jnp.int32, L)
                res = jnp.zeros((L,), jnp.float32)
                for j in range(L):
                    e = g * L + j
                    acc0 = jnp.zeros((L,), jnp.float32)
                    acc1 = jnp.zeros((L,), jnp.float32)
                    for t in range(0, d // L, 2):
                        acc0 = acc0 + qr[p][e, pl.ds(t * L, L)] * kr[p][e, pl.ds(t * L, L)]
                        acc1 = acc1 + qr[p][e, pl.ds((t + 1) * L, L)] * kr[p][e, pl.ds((t + 1) * L, L)]
                    res = jnp.where(lanes == j, jnp.sum(acc0 + acc1), res)
                sv[p][pl.ds(g * L, L)] = res
                return c2

            lax.fori_loop(0, C // L, group, 0)

        def chunk_body(c, p):
            wait_gather(p)

            @pl.when(c >= 2)
            def _():
                pltpu.make_async_copy(sv[p], s_hbm.at[pl.ds(base0, C)], sem_s[p]).wait()

            @pl.when(c + 2 < n_chunks)
            def _():
                issue_idx(c + 2, p)

            @pl.when(c + 1 < n_chunks)
            def _():
                wait_idx(1 - p)
                issue_gather(1 - p)

            compute(p)
            pltpu.async_copy(sv[p], s_hbm.at[pl.ds(base0 + c * C, C)], sem_s[p])

        issue_idx(0, 0)
        issue_idx(1, 1)
        wait_idx(0)
        issue_gather(0)

        def pair(j, carry):
            chunk_body(2 * j, 0)
            chunk_body(2 * j + 1, 1)
            return carry

        lax.fori_loop(0, n_pairs, pair, 0)
        pltpu.make_async_copy(sv[0], s_hbm.at[pl.ds(base0, C)], sem_s[0]).wait()
        pltpu.make_async_copy(sv[1], s_hbm.at[pl.ds(base0, C)], sem_s[1]).wait()

    return scores(q, k, row, col)


def _accum_sc(v, s, m, row, col):
    n_pad, d = v.shape
    e_pad = row.shape[0]
    per_s = e_pad // NS
    n_chunks = per_s // C
    n_pairs = n_chunks // 2
    rows_per_tile = n_pad // NS
    mesh = plsc.VectorSubcoreMesh(
        core_axis_name="c", subcore_axis_name="s", num_cores=NC, num_subcores=NS)

    @functools.partial(
        pl.kernel,
        out_type=jax.ShapeDtypeStruct((NC * n_pad, WCOL), jnp.float32),
        mesh=mesh,
        compiler_params=_SC_PARAMS,
        scratch_types=[
            [pltpu.VMEM((C,), jnp.int32)] * 2,
            [pltpu.VMEM((C,), jnp.int32)] * 2,
            [pltpu.VMEM((C,), jnp.int32)] * 2,
            [pltpu.VMEM((C,), jnp.float32)] * 2,
            pltpu.VMEM((C,), jnp.float32),
            [pltpu.VMEM((C, d), jnp.float32)] * 2,
            [pltpu.VMEM((C, WCOL), jnp.float32)] * 2,
            pltpu.VMEM((L,), jnp.float32),
            pltpu.VMEM_SHARED((n_pad, WCOL), jnp.float32),
            [pltpu.SemaphoreType.DMA] * 2,
            [pltpu.SemaphoreType.DMA] * 2,
            [pltpu.SemaphoreType.DMA] * 2,
            [pltpu.SemaphoreType.DMA] * 2,
            [pltpu.SemaphoreType.DMA] * 2,
        ],
    )
    def accum(v_hbm, s_hbm, m_hbm, row_hbm, col_hbm, out_hbm,
              row_v, col_v, rowsc, s_v, scmp, vrows, sbuf, m_v, hacc,
              sem_ir, sem_ic, sem_is, sem_v, sem_sc):
        cid = lax.axis_index("c")
        sid = lax.axis_index("s")

        def zrow(r, c0):
            for c9 in range(WCOL // L):
                sbuf[0][r, pl.ds(c9 * L, L)] = jnp.zeros((L,), jnp.float32)
            return c0

        lax.fori_loop(0, C, zrow, 0)
        for b in range(rows_per_tile // 128):
            pltpu.sync_copy(sbuf[0], hacc.at[pl.ds(sid * rows_per_tile + b * 128, 128)])
        plsc.subcore_barrier()

        pltpu.sync_copy(m_hbm.at[pl.ds(0, L)], m_v)
        mvec = m_v[...]
        dbase = cid * HW
        base0 = sid * per_s

        def issue_idx(c, p):
            base = base0 + c * C
            pltpu.async_copy(row_hbm.at[pl.ds(base, C)], row_v[p], sem_ir[p])
            pltpu.async_copy(col_hbm.at[pl.ds(base, C)], col_v[p], sem_ic[p])
            pltpu.async_copy(s_hbm.at[pl.ds(base, C)], s_v[p], sem_is[p])

        def chunk_body(c, p):
            pltpu.make_async_copy(v_hbm.at[col_v[p]], vrows[p], sem_v[p]).wait()
            pltpu.make_async_copy(row_hbm.at[pl.ds(0, C)], row_v[p], sem_ir[p]).wait()
            pltpu.make_async_copy(s_hbm.at[pl.ds(0, C)], s_v[p], sem_is[p]).wait()

            @pl.when(c >= 2)
            def _():
                pltpu.make_async_copy(sbuf[p], hacc.at[rowsc[p]], sem_sc[p]).wait()

            def stage(g, c0):
                rowsc[p][pl.ds(g * L, L)] = row_v[p][pl.ds(g * L, L)]
                scmp[pl.ds(g * L, L)] = s_v[p][pl.ds(g * L, L)]
                return c0

            lax.fori_loop(0, C // L, stage, 0)

            @pl.when(c + 2 < n_chunks)
            def _():
                issue_idx(c + 2, p)

            @pl.when(c + 1 < n_chunks)
            def _():
                pltpu.make_async_copy(col_hbm.at[pl.ds(0, C)], col_v[1 - p], sem_ic[1 - p]).wait()
                pltpu.async_copy(v_hbm.at[col_v[1 - p]], vrows[1 - p], sem_v[1 - p])

            def group(g, c0):
                s16 = scmp[pl.ds(g * L, L)]
                w16 = jnp.exp(s16 - mvec)
                for j in range(L):
                    e = g * L + j
                    wsp = jnp.full((L,), w16[j], jnp.float32)
                    for t in range(HW // L):
                        sbuf[p][e, pl.ds(t * L, L)] = (
                            vrows[p][e, pl.ds(dbase + t * L, L)] * wsp)
                    sbuf[p][e, pl.ds(HW, L)] = wsp
                return c0

            lax.fori_loop(0, C // L, group, 0)
            pltpu.async_copy(sbuf[p], hacc.at[rowsc[p]], sem_sc[p], add=True)

        issue_idx(0, 0)
        issue_idx(1, 1)
        pltpu.make_async_copy(col_hbm.at[pl.ds(0, C)], col_v[0], sem_ic[0]).wait()
        pltpu.async_copy(v_hbm.at[col_v[0]], vrows[0], sem_v[0])

        def pair(j, carry):
            chunk_body(2 * j, 0)
            chunk_body(2 * j + 1, 1)
            return carry

        lax.fori_loop(0, n_pairs, pair, 0)
        pltpu.make_async_copy(sbuf[0], hacc.at[rowsc[0]], sem_sc[0]).wait()
        pltpu.make_async_copy(sbuf[1], hacc.at[rowsc[1]], sem_sc[1]).wait()
        plsc.subcore_barrier()
        for b in range(rows_per_tile // 128):
            off = sid * rows_per_tile + b * 128
            pltpu.sync_copy(hacc.at[pl.ds(off, 128)],
                            out_hbm.at[pl.ds(cid * n_pad + off, 128)])

    return accum(v, s, m, row, col)


def _finalize_tc(acc, bias2d, n_pad):
    blk = 1024
    grid = n_pad // blk

    def body(a0_ref, a1_ref, b_ref, h_ref):
        a0 = a0_ref[...]
        a1 = a1_ref[...]
        h0 = a0[:, :HW] / a0[:, HW:HW + 1]
        h1 = a1[:, :HW] / a1[:, HW:HW + 1]
        h_ref[...] = jnp.concatenate([h0, h1], axis=1) + b_ref[...]

    return pl.pallas_call(
        body,
        grid=(grid,),
        in_specs=[
            pl.BlockSpec((blk, WCOL), lambda i: (i, 0)),
            pl.BlockSpec((blk, WCOL), lambda i, g=grid: (i + g, 0)),
            pl.BlockSpec((1, 128), lambda i: (0, 0)),
        ],
        out_specs=pl.BlockSpec((blk, 128), lambda i: (i, 0)),
        out_shape=jax.ShapeDtypeStruct((n_pad, 128), jnp.float32),
    )(acc, acc, bias2d)


def kernel(x, edge_index, query_kernel, query_bias, key_kernel, key_bias, kernel, bias):
    n, d = x.shape
    e = edge_index.shape[1]
    n_pad = -(-n // 2048) * 2048
    e_tot = e + n
    e_pad = -(-e_tot // (NW * C * 2)) * (NW * C * 2)

    loop = jnp.arange(n, dtype=jnp.int32)
    padidx = jnp.full((e_pad - e_tot,), n, dtype=jnp.int32)
    row = jnp.concatenate([edge_index[0], loop, padidx])
    col = jnp.concatenate([edge_index[1], loop, padidx])
    x_pad = jnp.pad(x, ((0, n_pad - n), (0, 0)))

    q, k, v = _qkv_tc(x_pad, query_kernel, query_bias.reshape(1, -1),
                      key_kernel, key_bias.reshape(1, -1), kernel)
    s = _scores_sc(q, k, row, col)
    m = _max_tc(s).reshape(-1)
    acc = _accum_sc(v, s, m, row, col)
    h = _finalize_tc(acc, bias.reshape(1, -1), n_pad)
    return h[:n]


# speedup vs baseline: 6.9278x; 1.0004x over previous
import functools

import jax
import jax.numpy as jnp
from jax import lax
from jax.experimental import pallas as pl
from jax.experimental.pallas import tpu as pltpu
from jax.experimental.pallas import tpu_sc as plsc

NC = 2
NS = 16
L = 16
NW = NC * NS
C = 128
HW = 64
WCOL = HW + L

_SC_PARAMS = pltpu.CompilerParams(
    needs_layout_passes=False, use_tc_tiling_on_sc=False)


def _qkv_tc(x_pad, wq, bq, wk, bk, wv):
    n_pad, d = x_pad.shape
    blk = 1024
    grid = n_pad // blk

    def body(x_ref, wq_ref, bq_ref, wk_ref, bk_ref, wv_ref, q_ref, k_ref, v_ref):
        xb = x_ref[...]
        q_ref[...] = jnp.maximum(
            jnp.dot(xb, wq_ref[...], preferred_element_type=jnp.float32) + bq_ref[...], 0.0)
        k_ref[...] = jnp.maximum(
            jnp.dot(xb, wk_ref[...], preferred_element_type=jnp.float32) + bk_ref[...], 0.0)
        v_ref[...] = jnp.dot(xb, wv_ref[...], preferred_element_type=jnp.float32)

    return pl.pallas_call(
        body,
        grid=(grid,),
        in_specs=[
            pl.BlockSpec((blk, d), lambda i: (i, 0)),
            pl.BlockSpec((d, d), lambda i: (0, 0)),
            pl.BlockSpec((1, d), lambda i: (0, 0)),
            pl.BlockSpec((d, d), lambda i: (0, 0)),
            pl.BlockSpec((1, d), lambda i: (0, 0)),
            pl.BlockSpec((d, d), lambda i: (0, 0)),
        ],
        out_specs=[pl.BlockSpec((blk, d), lambda i: (i, 0))] * 3,
        out_shape=[jax.ShapeDtypeStruct((n_pad, d), jnp.float32)] * 3,
    )(x_pad, wq, bq, wk, bk, wv)


def _max_tc(s):
    rows = s.shape[0] // 128
    s2 = s.reshape(rows, 128)

    def body(s_ref, m_ref):
        m_ref[...] = jnp.full((8, 128), jnp.max(s_ref[...]), jnp.float32)

    return pl.pallas_call(
        body, out_shape=jax.ShapeDtypeStruct((8, 128), jnp.float32))(s2)


def _scores_sc(q, k, row, col):
    n_pad, d = q.shape
    e_pad = row.shape[0]
    per_w = e_pad // NW
    n_chunks = per_w // C
    n_pairs = n_chunks // 2
    mesh = plsc.VectorSubcoreMesh(
        core_axis_name="c", subcore_axis_name="s", num_cores=NC, num_subcores=NS)

    @functools.partial(
        pl.kernel,
        out_type=jax.ShapeDtypeStruct((e_pad,), jnp.float32),
        mesh=mesh,
        compiler_params=_SC_PARAMS,
        scratch_types=[
            [pltpu.VMEM((C,), jnp.int32)] * 2,
            [pltpu.VMEM((C,), jnp.int32)] * 2,
            [pltpu.VMEM((C, d), jnp.float32)] * 2,
            [pltpu.VMEM((C, d), jnp.float32)] * 2,
            [pltpu.VMEM((C,), jnp.float32)] * 2,
            [pltpu.SemaphoreType.DMA] * 2,
            [pltpu.SemaphoreType.DMA] * 2,
            [pltpu.SemaphoreType.DMA] * 2,
            [pltpu.SemaphoreType.DMA] * 2,
            [pltpu.SemaphoreType.DMA] * 2,
        ],
    )
    def scores(q_hbm, k_hbm, row_hbm, col_hbm, s_hbm,
               row_v, col_v, qr, kr, sv, sem_ir, sem_ic, sem_q, sem_k, sem_s):
        wid = lax.axis_index("s") * NC + lax.axis_index("c")
        base0 = wid * per_w

        def issue_idx(c, p):
            base = base0 + c * C
            pltpu.async_copy(row_hbm.at[pl.ds(base, C)], row_v[p], sem_ir[p])
            pltpu.async_copy(col_hbm.at[pl.ds(base, C)], col_v[p], sem_ic[p])

        def wait_idx(p):
            pltpu.make_async_copy(row_hbm.at[pl.ds(0, C)], row_v[p], sem_ir[p]).wait()
            pltpu.make_async_copy(col_hbm.at[pl.ds(0, C)], col_v[p], sem_ic[p]).wait()

        def issue_gather(p):
            pltpu.async_copy(q_hbm.at[row_v[p]], qr[p], sem_q[p])
            pltpu.async_copy(k_hbm.at[col_v[p]], kr[p], sem_k[p])

        def wait_gather(p):
            pltpu.make_async_copy(q_hbm.at[row_v[p]], qr[p], sem_q[p]).wait()
            pltpu.make_async_copy(k_hbm.at[col_v[p]], kr[p], sem_k[p]).wait()

        def compute(p):
            def group(g, c2):
                lanes = lax.iota(jnp.int32, L)
                res = jnp.zeros((L,), jnp.float32)
                for j in range(0, L, 4):
                    accs = [[jnp.zeros((L,), jnp.float32)] * 2 for _ in range(4)]
                    for t in range(0, d // L, 2):
                        for u in range(4):
                            e = g * L + j + u
                            accs[u][0] = accs[u][0] + qr[p][e, pl.ds(t * L, L)] * kr[p][e, pl.ds(t * L, L)]
                            accs[u][1] = accs[u][1] + qr[p][e, pl.ds((t + 1) * L, L)] * kr[p][e, pl.ds((t + 1) * L, L)]
                    for u in range(4):
                        res = jnp.where(lanes == j + u, jnp.sum(accs[u][0] + accs[u][1]), res)
                sv[p][pl.ds(g * L, L)] = res
                return c2

            lax.fori_loop(0, C // L, group, 0)

        def chunk_body(c, p):
            wait_gather(p)

            @pl.when(c >= 2)
            def _():
                pltpu.make_async_copy(sv[p], s_hbm.at[pl.ds(base0, C)], sem_s[p]).wait()

            @pl.when(c + 2 < n_chunks)
            def _():
                issue_idx(c + 2, p)

            @pl.when(c + 1 < n_chunks)
            def _():
                wait_idx(1 - p)
                issue_gather(1 - p)

            compute(p)
            pltpu.async_copy(sv[p], s_hbm.at[pl.ds(base0 + c * C, C)], sem_s[p])

        issue_idx(0, 0)
        issue_idx(1, 1)
        wait_idx(0)
        issue_gather(0)

        def pair(j, carry):
            chunk_body(2 * j, 0)
            chunk_body(2 * j + 1, 1)
            return carry

        lax.fori_loop(0, n_pairs, pair, 0)
        pltpu.make_async_copy(sv[0], s_hbm.at[pl.ds(base0, C)], sem_s[0]).wait()
        pltpu.make_async_copy(sv[1], s_hbm.at[pl.ds(base0, C)], sem_s[1]).wait()

    return scores(q, k, row, col)


def _accum_sc(v, s, m, row, col):
    n_pad, d = v.shape
    e_pad = row.shape[0]
    per_s = e_pad // NS
    n_chunks = per_s // C
    n_pairs = n_chunks // 2
    rows_per_tile = n_pad // NS
    mesh = plsc.VectorSubcoreMesh(
        core_axis_name="c", subcore_axis_name="s", num_cores=NC, num_subcores=NS)

    @functools.partial(
        pl.kernel,
        out_type=jax.ShapeDtypeStruct((NC * n_pad, WCOL), jnp.float32),
        mesh=mesh,
        compiler_params=_SC_PARAMS,
        scratch_types=[
            [pltpu.VMEM((C,), jnp.int32)] * 2,
            [pltpu.VMEM((C,), jnp.int32)] * 2,
            [pltpu.VMEM((C,), jnp.int32)] * 2,
            [pltpu.VMEM((C,), jnp.float32)] * 2,
            pltpu.VMEM((C,), jnp.float32),
            [pltpu.VMEM((C, d), jnp.float32)] * 2,
            [pltpu.VMEM((C, WCOL), jnp.float32)] * 2,
            pltpu.VMEM((L,), jnp.float32),
            pltpu.VMEM_SHARED((n_pad, WCOL), jnp.float32),
            [pltpu.SemaphoreType.DMA] * 2,
            [pltpu.SemaphoreType.DMA] * 2,
            [pltpu.SemaphoreType.DMA] * 2,
            [pltpu.SemaphoreType.DMA] * 2,
            [pltpu.SemaphoreType.DMA] * 2,
        ],
    )
    def accum(v_hbm, s_hbm, m_hbm, row_hbm, col_hbm, out_hbm,
              row_v, col_v, rowsc, s_v, scmp, vrows, sbuf, m_v, hacc,
              sem_ir, sem_ic, sem_is, sem_v, sem_sc):
        cid = lax.axis_index("c")
        sid = lax.axis_index("s")

        def zrow(r, c0):
            for c9 in range(WCOL // L):
                sbuf[0][r, pl.ds(c9 * L, L)] = jnp.zeros((L,), jnp.float32)
            return c0

        lax.fori_loop(0, C, zrow, 0)
        for b in range(rows_per_tile // 128):
            pltpu.sync_copy(sbuf[0], hacc.at[pl.ds(sid * rows_per_tile + b * 128, 128)])
        plsc.subcore_barrier()

        pltpu.sync_copy(m_hbm.at[pl.ds(0, L)], m_v)
        mvec = m_v[...]
        dbase = cid * HW
        base0 = sid * per_s

        def issue_idx(c, p):
            base = base0 + c * C
            pltpu.async_copy(row_hbm.at[pl.ds(base, C)], row_v[p], sem_ir[p])
            pltpu.async_copy(col_hbm.at[pl.ds(base, C)], col_v[p], sem_ic[p])
            pltpu.async_copy(s_hbm.at[pl.ds(base, C)], s_v[p], sem_is[p])

        def chunk_body(c, p):
            pltpu.make_async_copy(v_hbm.at[col_v[p]], vrows[p], sem_v[p]).wait()
            pltpu.make_async_copy(row_hbm.at[pl.ds(0, C)], row_v[p], sem_ir[p]).wait()
            pltpu.make_async_copy(s_hbm.at[pl.ds(0, C)], s_v[p], sem_is[p]).wait()

            @pl.when(c >= 2)
            def _():
                pltpu.make_async_copy(sbuf[p], hacc.at[rowsc[p]], sem_sc[p]).wait()

            def stage(g, c0):
                rowsc[p][pl.ds(g * L, L)] = row_v[p][pl.ds(g * L, L)]
                scmp[pl.ds(g * L, L)] = s_v[p][pl.ds(g * L, L)]
                return c0

            lax.fori_loop(0, C // L, stage, 0)

            @pl.when(c + 2 < n_chunks)
            def _():
                issue_idx(c + 2, p)

            @pl.when(c + 1 < n_chunks)
            def _():
                pltpu.make_async_copy(col_hbm.at[pl.ds(0, C)], col_v[1 - p], sem_ic[1 - p]).wait()
                pltpu.async_copy(v_hbm.at[col_v[1 - p]], vrows[1 - p], sem_v[1 - p])

            def group(g, c0):
                s16 = scmp[pl.ds(g * L, L)]
                w16 = jnp.exp(s16 - mvec)
                for j in range(0, L, 4):
                    wsps = [jnp.full((L,), w16[j + u], jnp.float32) for u in range(4)]
                    for t in range(HW // L):
                        for u in range(4):
                            e = g * L + j + u
                            sbuf[p][e, pl.ds(t * L, L)] = (
                                vrows[p][e, pl.ds(dbase + t * L, L)] * wsps[u])
                    for u in range(4):
                        sbuf[p][g * L + j + u, pl.ds(HW, L)] = wsps[u]
                return c0

            lax.fori_loop(0, C // L, group, 0)
            pltpu.async_copy(sbuf[p], hacc.at[rowsc[p]], sem_sc[p], add=True)

        issue_idx(0, 0)
        issue_idx(1, 1)
        pltpu.make_async_copy(col_hbm.at[pl.ds(0, C)], col_v[0], sem_ic[0]).wait()
        pltpu.async_copy(v_hbm.at[col_v[0]], vrows[0], sem_v[0])

        def pair(j, carry):
            chunk_body(2 * j, 0)
            chunk_body(2 * j + 1, 1)
            return carry

        lax.fori_loop(0, n_pairs, pair, 0)
        pltpu.make_async_copy(sbuf[0], hacc.at[rowsc[0]], sem_sc[0]).wait()
        pltpu.make_async_copy(sbuf[1], hacc.at[rowsc[1]], sem_sc[1]).wait()
        plsc.subcore_barrier()
        for b in range(rows_per_tile // 128):
            off = sid * rows_per_tile + b * 128
            pltpu.sync_copy(hacc.at[pl.ds(off, 128)],
                            out_hbm.at[pl.ds(cid * n_pad + off, 128)])

    return accum(v, s, m, row, col)


def _finalize_tc(acc, bias2d, n_pad):
    blk = 1024
    grid = n_pad // blk

    def body(a0_ref, a1_ref, b_ref, h_ref):
        a0 = a0_ref[...]
        a1 = a1_ref[...]
        h0 = a0[:, :HW] / a0[:, HW:HW + 1]
        h1 = a1[:, :HW] / a1[:, HW:HW + 1]
        h_ref[...] = jnp.concatenate([h0, h1], axis=1) + b_ref[...]

    return pl.pallas_call(
        body,
        grid=(grid,),
        in_specs=[
            pl.BlockSpec((blk, WCOL), lambda i: (i, 0)),
            pl.BlockSpec((blk, WCOL), lambda i, g=grid: (i + g, 0)),
            pl.BlockSpec((1, 128), lambda i: (0, 0)),
        ],
        out_specs=pl.BlockSpec((blk, 128), lambda i: (i, 0)),
        out_shape=jax.ShapeDtypeStruct((n_pad, 128), jnp.float32),
    )(acc, acc, bias2d)


def kernel(x, edge_index, query_kernel, query_bias, key_kernel, key_bias, kernel, bias):
    n, d = x.shape
    e = edge_index.shape[1]
    n_pad = -(-n // 2048) * 2048
    e_tot = e + n
    e_pad = -(-e_tot // (NW * C * 2)) * (NW * C * 2)

    loop = jnp.arange(n, dtype=jnp.int32)
    padidx = jnp.full((e_pad - e_tot,), n, dtype=jnp.int32)
    row = jnp.concatenate([edge_index[0], loop, padidx])
    col = jnp.concatenate([edge_index[1], loop, padidx])
    x_pad = jnp.pad(x, ((0, n_pad - n), (0, 0)))

    q, k, v = _qkv_tc(x_pad, query_kernel, query_bias.reshape(1, -1),
                      key_kernel, key_bias.reshape(1, -1), kernel)
    s = _scores_sc(q, k, row, col)
    m = _max_tc(s).reshape(-1)
    acc = _accum_sc(v, s, m, row, col)
    h = _finalize_tc(acc, bias.reshape(1, -1), n_pad)
    return h[:n]


# speedup vs baseline: 8.1833x; 1.1812x over previous
import functools

import jax
import jax.numpy as jnp
from jax import lax
from jax.experimental import pallas as pl
from jax.experimental.pallas import tpu as pltpu
from jax.experimental.pallas import tpu_sc as plsc

NC = 2
NS = 16
L = 16
NW = NC * NS
C = 128
HW = 64
WCOL = HW + L

_SC_PARAMS = pltpu.CompilerParams(
    needs_layout_passes=False, use_tc_tiling_on_sc=False)


def _qkv_tc(x_pad, wq, bq, wk, bk, wv):
    n_pad, d = x_pad.shape
    blk = 1024
    grid = n_pad // blk

    def body(x_ref, wq_ref, bq_ref, wk_ref, bk_ref, wv_ref, q_ref, k_ref, v_ref):
        xb = x_ref[...]
        q_ref[...] = jnp.maximum(
            jnp.dot(xb, wq_ref[...], preferred_element_type=jnp.float32) + bq_ref[...], 0.0)
        k_ref[...] = jnp.maximum(
            jnp.dot(xb, wk_ref[...], preferred_element_type=jnp.float32) + bk_ref[...], 0.0)
        v_ref[...] = jnp.dot(xb, wv_ref[...], preferred_element_type=jnp.float32)

    return pl.pallas_call(
        body,
        grid=(grid,),
        in_specs=[
            pl.BlockSpec((blk, d), lambda i: (i, 0)),
            pl.BlockSpec((d, d), lambda i: (0, 0)),
            pl.BlockSpec((1, d), lambda i: (0, 0)),
            pl.BlockSpec((d, d), lambda i: (0, 0)),
            pl.BlockSpec((1, d), lambda i: (0, 0)),
            pl.BlockSpec((d, d), lambda i: (0, 0)),
        ],
        out_specs=[pl.BlockSpec((blk, d), lambda i: (i, 0))] * 3,
        out_shape=[jax.ShapeDtypeStruct((n_pad, d), jnp.float32)] * 3,
    )(x_pad, wq, bq, wk, bk, wv)


def _max_tc(s):
    rows = s.shape[0] // 128
    s2 = s.reshape(rows, 128)

    def body(s_ref, m_ref):
        m_ref[...] = jnp.full((8, 128), jnp.max(s_ref[...]), jnp.float32)

    return pl.pallas_call(
        body, out_shape=jax.ShapeDtypeStruct((8, 128), jnp.float32))(s2)


def _scores_sc(q, k, row, col):
    n_pad, d = q.shape
    e_pad = row.shape[0]
    per_w = e_pad // NW
    n_chunks = per_w // C
    n_pairs = n_chunks // 2
    mesh = plsc.VectorSubcoreMesh(
        core_axis_name="c", subcore_axis_name="s", num_cores=NC, num_subcores=NS)

    @functools.partial(
        pl.kernel,
        out_type=jax.ShapeDtypeStruct((e_pad,), jnp.float32),
        mesh=mesh,
        compiler_params=_SC_PARAMS,
        scratch_types=[
            [pltpu.VMEM((C,), jnp.int32)] * 2,
            [pltpu.VMEM((C,), jnp.int32)] * 2,
            [pltpu.VMEM((C, d), jnp.float32)] * 2,
            [pltpu.VMEM((C, d), jnp.float32)] * 2,
            [pltpu.VMEM((C,), jnp.float32)] * 2,
            [pltpu.SemaphoreType.DMA] * 2,
            [pltpu.SemaphoreType.DMA] * 2,
            [pltpu.SemaphoreType.DMA] * 2,
            [pltpu.SemaphoreType.DMA] * 2,
            [pltpu.SemaphoreType.DMA] * 2,
        ],
    )
    def scores(q_hbm, k_hbm, row_hbm, col_hbm, s_hbm,
               row_v, col_v, qr, kr, sv, sem_ir, sem_ic, sem_q, sem_k, sem_s):
        wid = lax.axis_index("s") * NC + lax.axis_index("c")
        base0 = wid * per_w

        def issue_idx(c, p):
            base = base0 + c * C
            pltpu.async_copy(row_hbm.at[pl.ds(base, C)], row_v[p], sem_ir[p])
            pltpu.async_copy(col_hbm.at[pl.ds(base, C)], col_v[p], sem_ic[p])

        def wait_idx(p):
            pltpu.make_async_copy(row_hbm.at[pl.ds(0, C)], row_v[p], sem_ir[p]).wait()
            pltpu.make_async_copy(col_hbm.at[pl.ds(0, C)], col_v[p], sem_ic[p]).wait()

        def issue_gather(p):
            pltpu.async_copy(q_hbm.at[row_v[p]], qr[p], sem_q[p])
            pltpu.async_copy(k_hbm.at[col_v[p]], kr[p], sem_k[p])

        def wait_gather(p):
            pltpu.make_async_copy(q_hbm.at[row_v[p]], qr[p], sem_q[p]).wait()
            pltpu.make_async_copy(k_hbm.at[col_v[p]], kr[p], sem_k[p]).wait()

        def compute(p):
            def group(g, c2):
                lanes = lax.iota(jnp.int32, L)
                res = jnp.zeros((L,), jnp.float32)
                for j in range(0, L, 4):
                    accs = [[jnp.zeros((L,), jnp.float32)] * 2 for _ in range(4)]
                    for t in range(0, d // L, 2):
                        for u in range(4):
                            e = g * L + j + u
                            accs[u][0] = accs[u][0] + qr[p][e, pl.ds(t * L, L)] * kr[p][e, pl.ds(t * L, L)]
                            accs[u][1] = accs[u][1] + qr[p][e, pl.ds((t + 1) * L, L)] * kr[p][e, pl.ds((t + 1) * L, L)]
                    for u in range(4):
                        res = jnp.where(lanes == j + u, jnp.sum(accs[u][0] + accs[u][1]), res)
                sv[p][pl.ds(g * L, L)] = res
                return c2

            lax.fori_loop(0, C // L, group, 0)

        def chunk_body(c, p):
            wait_gather(p)

            @pl.when(c >= 2)
            def _():
                pltpu.make_async_copy(sv[p], s_hbm.at[pl.ds(base0, C)], sem_s[p]).wait()

            @pl.when(c + 2 < n_chunks)
            def _():
                issue_idx(c + 2, p)

            @pl.when(c + 1 < n_chunks)
            def _():
                wait_idx(1 - p)
                issue_gather(1 - p)

            compute(p)
            pltpu.async_copy(sv[p], s_hbm.at[pl.ds(base0 + c * C, C)], sem_s[p])

        issue_idx(0, 0)
        issue_idx(1, 1)
        wait_idx(0)
        issue_gather(0)

        def pair(j, carry):
            chunk_body(2 * j, 0)
            chunk_body(2 * j + 1, 1)
            return carry

        lax.fori_loop(0, n_pairs, pair, 0)
        pltpu.make_async_copy(sv[0], s_hbm.at[pl.ds(base0, C)], sem_s[0]).wait()
        pltpu.make_async_copy(sv[1], s_hbm.at[pl.ds(base0, C)], sem_s[1]).wait()

    return scores(q, k, row, col)


def _accum_sc(v0, v1, s, m, row, col):
    n_pad = v0.shape[0]
    e_pad = row.shape[0]
    per_s = e_pad // NS
    n_chunks = per_s // C
    n_pairs = n_chunks // 2
    rows_per_tile = n_pad // NS
    mesh = plsc.VectorSubcoreMesh(
        core_axis_name="c", subcore_axis_name="s", num_cores=NC, num_subcores=NS)

    @functools.partial(
        pl.kernel,
        out_type=jax.ShapeDtypeStruct((NC * n_pad, WCOL), jnp.float32),
        mesh=mesh,
        compiler_params=_SC_PARAMS,
        scratch_types=[
            [pltpu.VMEM((C,), jnp.int32)] * 2,
            [pltpu.VMEM((C,), jnp.int32)] * 2,
            [pltpu.VMEM((C,), jnp.int32)] * 2,
            [pltpu.VMEM((C,), jnp.float32)] * 2,
            pltpu.VMEM((C,), jnp.float32),
            [pltpu.VMEM((C, HW), jnp.float32)] * 2,
            [pltpu.VMEM((C, WCOL), jnp.float32)] * 2,
            pltpu.VMEM((L,), jnp.float32),
            pltpu.VMEM_SHARED((n_pad, WCOL), jnp.float32),
            [pltpu.SemaphoreType.DMA] * 2,
            [pltpu.SemaphoreType.DMA] * 2,
            [pltpu.SemaphoreType.DMA] * 2,
            [pltpu.SemaphoreType.DMA] * 2,
            [pltpu.SemaphoreType.DMA] * 2,
        ],
    )
    def accum(v0_hbm, v1_hbm, s_hbm, m_hbm, row_hbm, col_hbm, out_hbm,
              row_v, col_v, rowsc, s_v, scmp, vrows, sbuf, m_v, hacc,
              sem_ir, sem_ic, sem_is, sem_v, sem_sc):
        cid = lax.axis_index("c")
        sid = lax.axis_index("s")

        def issue_vgather(p):
            @pl.when(cid == 0)
            def _():
                pltpu.async_copy(v0_hbm.at[col_v[p]], vrows[p], sem_v[p])

            @pl.when(cid == 1)
            def _():
                pltpu.async_copy(v1_hbm.at[col_v[p]], vrows[p], sem_v[p])

        def zrow(r, c0):
            for c9 in range(WCOL // L):
                sbuf[0][r, pl.ds(c9 * L, L)] = jnp.zeros((L,), jnp.float32)
            return c0

        lax.fori_loop(0, C, zrow, 0)
        for b in range(rows_per_tile // 128):
            pltpu.sync_copy(sbuf[0], hacc.at[pl.ds(sid * rows_per_tile + b * 128, 128)])
        plsc.subcore_barrier()

        pltpu.sync_copy(m_hbm.at[pl.ds(0, L)], m_v)
        mvec = m_v[...]
        base0 = sid * per_s

        def issue_idx(c, p):
            base = base0 + c * C
            pltpu.async_copy(row_hbm.at[pl.ds(base, C)], row_v[p], sem_ir[p])
            pltpu.async_copy(col_hbm.at[pl.ds(base, C)], col_v[p], sem_ic[p])
            pltpu.async_copy(s_hbm.at[pl.ds(base, C)], s_v[p], sem_is[p])

        def chunk_body(c, p):
            pltpu.make_async_copy(v0_hbm.at[col_v[p]], vrows[p], sem_v[p]).wait()
            pltpu.make_async_copy(row_hbm.at[pl.ds(0, C)], row_v[p], sem_ir[p]).wait()
            pltpu.make_async_copy(s_hbm.at[pl.ds(0, C)], s_v[p], sem_is[p]).wait()

            @pl.when(c >= 2)
            def _():
                pltpu.make_async_copy(sbuf[p], hacc.at[rowsc[p]], sem_sc[p]).wait()

            def stage(g, c0):
                rowsc[p][pl.ds(g * L, L)] = row_v[p][pl.ds(g * L, L)]
                scmp[pl.ds(g * L, L)] = s_v[p][pl.ds(g * L, L)]
                return c0

            lax.fori_loop(0, C // L, stage, 0)

            @pl.when(c + 2 < n_chunks)
            def _():
                issue_idx(c + 2, p)

            @pl.when(c + 1 < n_chunks)
            def _():
                pltpu.make_async_copy(col_hbm.at[pl.ds(0, C)], col_v[1 - p], sem_ic[1 - p]).wait()
                issue_vgather(1 - p)

            def group(g, c0):
                s16 = scmp[pl.ds(g * L, L)]
                w16 = jnp.exp(s16 - mvec)
                for j in range(0, L, 4):
                    wsps = [jnp.full((L,), w16[j + u], jnp.float32) for u in range(4)]
                    for t in range(HW // L):
                        for u in range(4):
                            e = g * L + j + u
                            sbuf[p][e, pl.ds(t * L, L)] = (
                                vrows[p][e, pl.ds(t * L, L)] * wsps[u])
                    for u in range(4):
                        sbuf[p][g * L + j + u, pl.ds(HW, L)] = wsps[u]
                return c0

            lax.fori_loop(0, C // L, group, 0)
            pltpu.async_copy(sbuf[p], hacc.at[rowsc[p]], sem_sc[p], add=True)

        issue_idx(0, 0)
        issue_idx(1, 1)
        pltpu.make_async_copy(col_hbm.at[pl.ds(0, C)], col_v[0], sem_ic[0]).wait()
        issue_vgather(0)

        def pair(j, carry):
            chunk_body(2 * j, 0)
            chunk_body(2 * j + 1, 1)
            return carry

        lax.fori_loop(0, n_pairs, pair, 0)
        pltpu.make_async_copy(sbuf[0], hacc.at[rowsc[0]], sem_sc[0]).wait()
        pltpu.make_async_copy(sbuf[1], hacc.at[rowsc[1]], sem_sc[1]).wait()
        plsc.subcore_barrier()
        for b in range(rows_per_tile // 128):
            off = sid * rows_per_tile + b * 128
            pltpu.sync_copy(hacc.at[pl.ds(off, 128)],
                            out_hbm.at[pl.ds(cid * n_pad + off, 128)])

    return accum(v0, v1, s, m, row, col)


def _finalize_tc(acc, bias2d, n_pad):
    blk = 1024
    grid = n_pad // blk

    def body(a0_ref, a1_ref, b_ref, h_ref):
        a0 = a0_ref[...]
        a1 = a1_ref[...]
        h0 = a0[:, :HW] / a0[:, HW:HW + 1]
        h1 = a1[:, :HW] / a1[:, HW:HW + 1]
        h_ref[...] = jnp.concatenate([h0, h1], axis=1) + b_ref[...]

    return pl.pallas_call(
        body,
        grid=(grid,),
        in_specs=[
            pl.BlockSpec((blk, WCOL), lambda i: (i, 0)),
            pl.BlockSpec((blk, WCOL), lambda i, g=grid: (i + g, 0)),
            pl.BlockSpec((1, 128), lambda i: (0, 0)),
        ],
        out_specs=pl.BlockSpec((blk, 128), lambda i: (i, 0)),
        out_shape=jax.ShapeDtypeStruct((n_pad, 128), jnp.float32),
    )(acc, acc, bias2d)


def kernel(x, edge_index, query_kernel, query_bias, key_kernel, key_bias, kernel, bias):
    n, d = x.shape
    e = edge_index.shape[1]
    n_pad = -(-n // 2048) * 2048
    e_tot = e + n
    e_pad = -(-e_tot // (NW * C * 2)) * (NW * C * 2)

    loop = jnp.arange(n, dtype=jnp.int32)
    padidx = jnp.full((e_pad - e_tot,), n, dtype=jnp.int32)
    row = jnp.concatenate([edge_index[0], loop, padidx])
    col = jnp.concatenate([edge_index[1], loop, padidx])
    x_pad = jnp.pad(x, ((0, n_pad - n), (0, 0)))

    q, k, v = _qkv_tc(x_pad, query_kernel, query_bias.reshape(1, -1),
                      key_kernel, key_bias.reshape(1, -1), kernel)
    s = _scores_sc(q, k, row, col)
    m = _max_tc(s).reshape(-1)
    acc = _accum_sc(v[:, :HW], v[:, HW:], s, m, row, col)
    h = _finalize_tc(acc, bias.reshape(1, -1), n_pad)
    return h[:n]
